# baseline (speedup 1.0000x reference)
import jax
import jax.numpy as jnp
from jax import lax
from jax.experimental import pallas as pl
from jax.experimental.pallas import tpu as pltpu
from jax.experimental.pallas import tpu_sc as plsc

_NUM_NODES = 1000000
_NUM_CHANNELS = 32
_BATCH = 16384

_INFO = plsc.get_sparse_core_info()
_NC = _INFO.num_cores
_NS = _INFO.num_subcores
_NW = _NC * _NS
_B_PER_W = _BATCH // _NW
_G = 16
_NGROUPS = _B_PER_W // _G


def _make_gather():
    mesh = plsc.VectorSubcoreMesh(core_axis_name="c", subcore_axis_name="s")

    @pl.kernel(
        mesh=mesh,
        out_type=jax.ShapeDtypeStruct((_NUM_CHANNELS, _BATCH), jnp.float32),
        scratch_types=[
            pltpu.VMEM((_B_PER_W,), jnp.int32),
            pltpu.VMEM((_G, _NUM_CHANNELS, 128), jnp.float32),
            pltpu.VMEM((_NUM_CHANNELS, _B_PER_W), jnp.float32),
            pltpu.SemaphoreType.DMA,
        ],
        compiler_params=pltpu.CompilerParams(needs_layout_passes=False),
    )
    def gather(idx_hbm, table_hbm, out_hbm, idx_v, ring, out_v, sem):
        wid = lax.axis_index("s") * _NC + lax.axis_index("c")
        base = wid * _B_PER_W
        pltpu.sync_copy(idx_hbm.at[pl.ds(base, _B_PER_W)], idx_v)
        iota = lax.iota(jnp.int32, 16)

        def group(g, _):
            nv = idx_v[pl.ds(g * _G, _G)]
            copies = []
            for j in range(_G):
                n = jnp.sum(jnp.where(iota == j, nv, 0))
                gbase = pl.multiple_of((n >> 7) * 128, 128)
                for o in range(_NUM_CHANNELS // 8):
                    copies.append(
                        pltpu.async_copy(
                            table_hbm.at[pl.ds(o * 8, 8), pl.ds(gbase, 128)],
                            ring.at[j, pl.ds(o * 8, 8)],
                            sem,
                        )
                    )
            for cp in copies:
                cp.wait()
            colv = idx_v[pl.ds(g * _G, _G)] & 127
            for c in range(_NUM_CHANNELS):
                cv = jnp.full((16,), c, jnp.int32)
                v = plsc.load_gather(ring, [iota, cv, colv])
                out_v[c, pl.ds(g * _G, _G)] = v
            return _

        lax.fori_loop(0, _NGROUPS, group, None)
        pltpu.sync_copy(out_v, out_hbm.at[:, pl.ds(base, _B_PER_W)])

    return gather


_gather = _make_gather()


@jax.jit
def kernel(x_indices, embedding_weight):
    idx = x_indices.astype(jnp.int32)
    table_t = embedding_weight.T
    out_t = _gather(idx, table_t)
    return out_t.T


# speedup vs baseline: 1.0876x; 1.0876x over previous
import jax
import jax.numpy as jnp
from jax import lax
from jax.experimental import pallas as pl
from jax.experimental.pallas import tpu as pltpu
from jax.experimental.pallas import tpu_sc as plsc

_NUM_NODES = 1000000
_NUM_CHANNELS = 32
_BATCH = 16384

_INFO = plsc.get_sparse_core_info()
_NC = _INFO.num_cores
_NS = _INFO.num_subcores
_NW = _NC * _NS
_B_PER_W = _BATCH // _NW
_G = 16
_NGROUPS = _B_PER_W // _G


def _make_gather():
    mesh = plsc.VectorSubcoreMesh(core_axis_name="c", subcore_axis_name="s")

    @pl.kernel(
        mesh=mesh,
        out_type=jax.ShapeDtypeStruct((_NUM_CHANNELS, _BATCH), jnp.float32),
        scratch_types=[
            pltpu.VMEM((_B_PER_W,), jnp.int32),
            pltpu.VMEM((_G, _NUM_CHANNELS, 128), jnp.float32),
            pltpu.VMEM((_NUM_CHANNELS, _B_PER_W), jnp.float32),
            pltpu.SemaphoreType.DMA((_G,)),
        ],
        compiler_params=pltpu.CompilerParams(needs_layout_passes=False),
    )
    def gather(idx_hbm, table_hbm, out_hbm, idx_v, ring, out_v, sems):
        wid = lax.axis_index("s") * _NC + lax.axis_index("c")
        base = wid * _B_PER_W
        pltpu.sync_copy(idx_hbm.at[pl.ds(base, _B_PER_W)], idx_v)
        iota = lax.iota(jnp.int32, 16)

        def extract(g, j):
            nv = idx_v[pl.ds(g * _G, _G)]
            return jnp.sum(jnp.where(iota == j, nv, 0))

        def fire(g, j):
            n = extract(g, j)
            gbase = pl.multiple_of((n >> 7) * 128, 128)
            for o in range(_NUM_CHANNELS // 8):
                pltpu.async_copy(
                    table_hbm.at[pl.ds(o * 8, 8), pl.ds(gbase, 128)],
                    ring.at[j, pl.ds(o * 8, 8)],
                    sems.at[j],
                )

        for j in range(_G):
            fire(0, j)

        def group(g, _):
            for j in range(_G):
                pltpu.make_async_copy(
                    table_hbm.at[:, pl.ds(0, 128)], ring.at[j], sems.at[j]
                ).wait()
                col = extract(g, j) & 127
                cv = jnp.full((16,), col, jnp.int32)
                jv = jnp.full((16,), j, jnp.int32)
                i = g * _G + j
                iv = jnp.full((16,), i, jnp.int32)
                v0 = plsc.load_gather(ring, [jv, iota, cv])
                v1 = plsc.load_gather(ring, [jv, iota + 16, cv])
                plsc.store_scatter(out_v, [iota, iv], v0)
                plsc.store_scatter(out_v, [iota + 16, iv], v1)

                @pl.when(g < _NGROUPS - 1)
                def _():
                    fire(g + 1, j)

            return _

        lax.fori_loop(0, _NGROUPS, group, None)
        pltpu.sync_copy(out_v, out_hbm.at[:, pl.ds(base, _B_PER_W)])

    return gather


_gather = _make_gather()


@jax.jit
def kernel(x_indices, embedding_weight):
    idx = x_indices.astype(jnp.int32)
    table_t = embedding_weight.T
    out_t = _gather(idx, table_t)
    return out_t.T
